# baseline (speedup 1.0000x reference)
import functools

import jax
import jax.numpy as jnp
from jax import lax
from jax.experimental import pallas as pl
from jax.experimental.pallas import tpu as pltpu
from jax.experimental.pallas import tpu_sc as plsc

BATCH = 16384
EMB = 128
EPS = 1e-5

NC = 2
NS = 16
NW = NC * NS
CH = 128

NSPLIT = 2
CS = BATCH // NSPLIT
BB = 2048


def _make_gather_body(cs, chunk):
    rpw = cs // NW
    nch = rpw // CH

    def body(uid_hbm, cid_hbm, utab_hbm, ctab_hbm, ue_hbm, ce_hbm,
             idx_v, rows_a, rows_b, sem_a, sem_b):
        wid = lax.axis_index("s") * NC + lax.axis_index("c")
        base = wid * rpw
        for tab_hbm, ids_hbm, out_hbm in ((utab_hbm, uid_hbm, ue_hbm),
                                          (ctab_hbm, cid_hbm, ce_hbm)):
            for j in range(nch):
                pltpu.sync_copy(
                    ids_hbm.at[pl.ds(chunk * cs + base + j * CH, CH)],
                    idx_v.at[j])
            pltpu.async_copy(tab_hbm.at[idx_v.at[0]], rows_a, sem_a)
            for j in range(nch):
                cur, nxt = (rows_a, rows_b) if j % 2 == 0 else (rows_b, rows_a)
                cur_s, nxt_s = (sem_a, sem_b) if j % 2 == 0 else (sem_b, sem_a)
                if j + 1 < nch:
                    pltpu.async_copy(tab_hbm.at[idx_v.at[j + 1]], nxt, nxt_s)
                pltpu.make_async_copy(tab_hbm.at[idx_v.at[j]], cur, cur_s).wait()
                pltpu.sync_copy(cur, out_hbm.at[pl.ds(base + j * CH, CH)])

    return body


@functools.cache
def _gather(cs, chunk):
    nch = cs // NW // CH
    return pl.kernel(
        _make_gather_body(cs, chunk),
        out_type=(jax.ShapeDtypeStruct((cs, EMB), jnp.float32),
                  jax.ShapeDtypeStruct((cs, EMB), jnp.float32)),
        mesh=plsc.VectorSubcoreMesh(core_axis_name="c", subcore_axis_name="s",
                                    num_cores=NC, num_subcores=NS),
        scratch_types=(pltpu.VMEM((nch, CH), jnp.int32),
                       pltpu.VMEM((CH, EMB), jnp.float32),
                       pltpu.VMEM((CH, EMB), jnp.float32),
                       pltpu.SemaphoreType.DMA,
                       pltpu.SemaphoreType.DMA),
    )


_DN_T = (((1,), (1,)), ((), ()))


def _mlp_body(ue, ce, w0, b0, g0, beta0, rm0, rv0,
              w1, b1, g1, beta1, rm1, rv1,
              w2, b2, g2, beta2, rm2, rv2,
              w3, b3, out):
    bf = jnp.bfloat16
    w0v = w0[...].astype(bf)
    z0 = (lax.dot_general(ue[...].astype(bf), w0v[:, :EMB], _DN_T,
                          preferred_element_type=jnp.float32)
          + lax.dot_general(ce[...].astype(bf), w0v[:, EMB:], _DN_T,
                            preferred_element_type=jnp.float32)
          + b0[...])
    s0 = g0[...] * lax.rsqrt(rv0[...] + EPS)
    y0 = jnp.maximum(z0, 0.0) * s0 + (beta0[...] - rm0[...] * s0)
    z1 = lax.dot_general(y0.astype(bf), w1[...].astype(bf), _DN_T,
                         preferred_element_type=jnp.float32) + b1[...]
    s1 = g1[...] * lax.rsqrt(rv1[...] + EPS)
    y1 = jnp.maximum(z1, 0.0) * s1 + (beta1[...] - rm1[...] * s1)
    z2 = lax.dot_general(y1.astype(bf), w2[...].astype(bf), _DN_T,
                         preferred_element_type=jnp.float32) + b2[...]
    s2 = g2[...] * lax.rsqrt(rv2[...] + EPS)
    y2 = jnp.maximum(z2, 0.0) * s2 + (beta2[...] - rm2[...] * s2)
    z3 = jnp.sum(y2 * w3[...], axis=1) + b3[0, 0]
    out[...] = 1.0 / (1.0 + jnp.exp(-z3))


def _full(shape):
    return pl.BlockSpec(shape, lambda i: (0,) * len(shape))


@functools.cache
def _mlp(cs):
    return pl.pallas_call(
        _mlp_body,
        grid=(cs // BB,),
        in_specs=[
            pl.BlockSpec((BB, EMB), lambda i: (i, 0)),
            pl.BlockSpec((BB, EMB), lambda i: (i, 0)),
            _full((256, 256)), _full((1, 256)), _full((1, 256)),
            _full((1, 256)), _full((1, 256)), _full((1, 256)),
            _full((128, 256)), _full((1, 128)), _full((1, 128)),
            _full((1, 128)), _full((1, 128)), _full((1, 128)),
            _full((64, 128)), _full((1, 64)), _full((1, 64)),
            _full((1, 64)), _full((1, 64)), _full((1, 64)),
            _full((1, 64)), _full((1, 1)),
        ],
        out_specs=pl.BlockSpec((BB,), lambda i: (i,)),
        out_shape=jax.ShapeDtypeStruct((cs,), jnp.float32),
        compiler_params=pltpu.CompilerParams(
            dimension_semantics=("arbitrary",)),
    )


def kernel(user_ids, content_ids, user_table, content_table,
           W0, b0, g0, beta0, rm0, rv0,
           W1, b1, g1, beta1, rm1, rv1,
           W2, b2, g2, beta2, rm2, rv2,
           W3, b3):
    uid = user_ids.astype(jnp.int32)
    cid = content_ids.astype(jnp.int32)
    r = lambda v: v.reshape(1, -1)
    mlp_args = (W0, r(b0), r(g0), r(beta0), r(rm0), r(rv0),
                W1, r(b1), r(g1), r(beta1), r(rm1), r(rv1),
                W2, r(b2), r(g2), r(beta2), r(rm2), r(rv2),
                W3, b3.reshape(1, 1))
    scores = []
    for c in range(NSPLIT):
        ue, ce = _gather(CS, c)(uid, cid, user_table, content_table)
        scores.append(_mlp(CS)(ue, ce, *mlp_args))
    return jnp.concatenate(scores) if NSPLIT > 1 else scores[0]


# speedup vs baseline: 1.0065x; 1.0065x over previous
import functools

import jax
import jax.numpy as jnp
from jax import lax
from jax.experimental import pallas as pl
from jax.experimental.pallas import tpu as pltpu
from jax.experimental.pallas import tpu_sc as plsc

BATCH = 16384
EMB = 128
EPS = 1e-5

NC = 2
NS = 16
NW = NC * NS
CH = 128

NSPLIT = 2
CS = BATCH // NSPLIT
BB = 2048


def _make_gather_body(cs, chunk):
    rpw = cs // NW
    nch = rpw // CH

    def body(uid_hbm, cid_hbm, utab_hbm, ctab_hbm, ue_hbm, ce_hbm,
             idx_v, rows_a, rows_b, sem_a, sem_b):
        wid = lax.axis_index("s") * NC + lax.axis_index("c")
        base = wid * rpw
        for tab_hbm, ids_hbm, out_hbm in ((utab_hbm, uid_hbm, ue_hbm),
                                          (ctab_hbm, cid_hbm, ce_hbm)):
            for j in range(nch):
                pltpu.sync_copy(
                    ids_hbm.at[pl.ds(chunk * cs + base + j * CH, CH)],
                    idx_v.at[j])
            pltpu.async_copy(tab_hbm.at[idx_v.at[0]], rows_a, sem_a)
            for j in range(nch):
                cur, nxt = (rows_a, rows_b) if j % 2 == 0 else (rows_b, rows_a)
                cur_s, nxt_s = (sem_a, sem_b) if j % 2 == 0 else (sem_b, sem_a)
                if j + 1 < nch:
                    pltpu.async_copy(tab_hbm.at[idx_v.at[j + 1]], nxt, nxt_s)
                pltpu.make_async_copy(tab_hbm.at[idx_v.at[j]], cur, cur_s).wait()
                pltpu.sync_copy(cur, out_hbm.at[pl.ds(base + j * CH, CH)])

    return body


@functools.cache
def _gather(cs, chunk):
    nch = cs // NW // CH
    return pl.kernel(
        _make_gather_body(cs, chunk),
        out_type=(jax.ShapeDtypeStruct((cs, EMB), jnp.float32),
                  jax.ShapeDtypeStruct((cs, EMB), jnp.float32)),
        mesh=plsc.VectorSubcoreMesh(core_axis_name="c", subcore_axis_name="s",
                                    num_cores=NC, num_subcores=NS),
        scratch_types=(pltpu.VMEM((nch, CH), jnp.int32),
                       pltpu.VMEM((CH, EMB), jnp.float32),
                       pltpu.VMEM((CH, EMB), jnp.float32),
                       pltpu.SemaphoreType.DMA,
                       pltpu.SemaphoreType.DMA),
    )


_DN_T = (((1,), (1,)), ((), ()))


def _mlp_body(ue, ce, w0, w1, w2, p0, p1, p2, out):
    bf = jnp.bfloat16
    w0v = w0[...].astype(bf)
    p0v, p1v, p2v = p0[...], p1[...], p2[...]
    z0 = (lax.dot_general(ue[...].astype(bf), w0v[:, :EMB], _DN_T,
                          preferred_element_type=jnp.float32)
          + lax.dot_general(ce[...].astype(bf), w0v[:, EMB:], _DN_T,
                            preferred_element_type=jnp.float32)
          + p0v[0:1])
    y0 = jnp.maximum(z0, 0.0) * p0v[1:2] + p0v[2:3]
    z1 = lax.dot_general(y0.astype(bf), w1[...].astype(bf), _DN_T,
                         preferred_element_type=jnp.float32) + p1v[0:1]
    y1 = jnp.maximum(z1, 0.0) * p1v[1:2] + p1v[2:3]
    z2 = lax.dot_general(y1.astype(bf), w2[...].astype(bf), _DN_T,
                         preferred_element_type=jnp.float32) + p2v[0:1]
    y2 = jnp.maximum(z2, 0.0) * p2v[1:2] + p2v[2:3]
    z3 = jnp.sum(y2 * p2v[3:4], axis=1) + p2v[4, 0]
    out[...] = 1.0 / (1.0 + jnp.exp(-z3))


def _full(shape):
    return pl.BlockSpec(shape, lambda i: (0,) * len(shape))


@functools.cache
def _mlp(cs):
    return pl.pallas_call(
        _mlp_body,
        grid=(cs // BB,),
        in_specs=[
            pl.BlockSpec((BB, EMB), lambda i: (i, 0)),
            pl.BlockSpec((BB, EMB), lambda i: (i, 0)),
            _full((256, 256)), _full((128, 256)), _full((64, 128)),
            _full((3, 256)), _full((3, 128)), _full((5, 64)),
        ],
        out_specs=pl.BlockSpec((BB,), lambda i: (i,)),
        out_shape=jax.ShapeDtypeStruct((cs,), jnp.float32),
        compiler_params=pltpu.CompilerParams(
            dimension_semantics=("arbitrary",)),
    )


def kernel(user_ids, content_ids, user_table, content_table,
           W0, b0, g0, beta0, rm0, rv0,
           W1, b1, g1, beta1, rm1, rv1,
           W2, b2, g2, beta2, rm2, rv2,
           W3, b3):
    uid = user_ids.astype(jnp.int32)
    cid = content_ids.astype(jnp.int32)
    s0 = g0 * lax.rsqrt(rv0 + EPS)
    s1 = g1 * lax.rsqrt(rv1 + EPS)
    s2 = g2 * lax.rsqrt(rv2 + EPS)
    p0 = jnp.stack([b0, s0, beta0 - rm0 * s0])
    p1 = jnp.stack([b1, s1, beta1 - rm1 * s1])
    p2 = jnp.stack([b2, s2, beta2 - rm2 * s2, W3[0],
                    jnp.broadcast_to(b3, (64,))])
    mlp_args = (W0, W1, W2, p0, p1, p2)
    scores = []
    for c in range(NSPLIT):
        ue, ce = _gather(CS, c)(uid, cid, user_table, content_table)
        scores.append(_mlp(CS)(ue, ce, *mlp_args))
    return jnp.concatenate(scores) if NSPLIT > 1 else scores[0]


# speedup vs baseline: 1.0106x; 1.0040x over previous
import functools

import jax
import jax.numpy as jnp
from jax import lax
from jax.experimental import pallas as pl
from jax.experimental.pallas import tpu as pltpu
from jax.experimental.pallas import tpu_sc as plsc

BATCH = 16384
EMB = 128
EPS = 1e-5

NC = 2
NS = 16
NW = NC * NS
CH = 128

NSPLIT = 2
CS = BATCH // NSPLIT
BB = 2048


def _make_gather_body(cs, chunk):
    rpw = cs // NW
    nch = rpw // CH

    def body(uid_hbm, cid_hbm, utab_hbm, ctab_hbm, ue_hbm, ce_hbm,
             idx_v, rows_a, rows_b, sem_a, sem_b):
        wid = lax.axis_index("s") * NC + lax.axis_index("c")
        base = wid * rpw
        for tab_hbm, ids_hbm, out_hbm in ((utab_hbm, uid_hbm, ue_hbm),
                                          (ctab_hbm, cid_hbm, ce_hbm)):
            for j in range(nch):
                pltpu.sync_copy(
                    ids_hbm.at[pl.ds(chunk * cs + base + j * CH, CH)],
                    idx_v.at[j])
            pltpu.async_copy(tab_hbm.at[idx_v.at[0]], rows_a, sem_a)
            for j in range(nch):
                cur, nxt = (rows_a, rows_b) if j % 2 == 0 else (rows_b, rows_a)
                cur_s, nxt_s = (sem_a, sem_b) if j % 2 == 0 else (sem_b, sem_a)
                if j + 1 < nch:
                    pltpu.async_copy(tab_hbm.at[idx_v.at[j + 1]], nxt, nxt_s)
                pltpu.make_async_copy(tab_hbm.at[idx_v.at[j]], cur, cur_s).wait()
                pltpu.sync_copy(cur, out_hbm.at[pl.ds(base + j * CH, CH)])

    return body


@functools.cache
def _gather(cs, chunk):
    nch = cs // NW // CH
    return pl.kernel(
        _make_gather_body(cs, chunk),
        out_type=(jax.ShapeDtypeStruct((cs, EMB), jnp.float32),
                  jax.ShapeDtypeStruct((cs, EMB), jnp.float32)),
        mesh=plsc.VectorSubcoreMesh(core_axis_name="c", subcore_axis_name="s",
                                    num_cores=NC, num_subcores=NS),
        scratch_types=(pltpu.VMEM((nch, CH), jnp.int32),
                       pltpu.VMEM((CH, EMB), jnp.float32),
                       pltpu.VMEM((CH, EMB), jnp.float32),
                       pltpu.SemaphoreType.DMA,
                       pltpu.SemaphoreType.DMA),
    )


_DN_T = (((1,), (1,)), ((), ()))


def _mlp_body(ue, ce, w0, w1, w2, w3c, p0, p1, p2, out):
    bf = jnp.bfloat16
    w0v = w0[...].astype(bf)
    p0v, p1v, p2v = p0[...], p1[...], p2[...]
    z0 = (lax.dot_general(ue[...].astype(bf), w0v[:, :EMB], _DN_T,
                          preferred_element_type=jnp.float32)
          + lax.dot_general(ce[...].astype(bf), w0v[:, EMB:], _DN_T,
                            preferred_element_type=jnp.float32)
          + p0v[0:1])
    y0 = jnp.maximum(z0, 0.0) * p0v[1:2] + p0v[2:3]
    z1 = lax.dot_general(y0.astype(bf), w1[...].astype(bf), _DN_T,
                         preferred_element_type=jnp.float32) + p1v[0:1]
    y1 = jnp.maximum(z1, 0.0) * p1v[1:2] + p1v[2:3]
    z2 = lax.dot_general(y1.astype(bf), w2[...].astype(bf), _DN_T,
                         preferred_element_type=jnp.float32) + p2v[0:1]
    y2 = jnp.maximum(z2, 0.0) * p2v[1:2] + p2v[2:3]
    z3 = lax.dot_general(y2.astype(bf), w3c[...].astype(bf),
                         (((1,), (0,)), ((), ())),
                         preferred_element_type=jnp.float32) + p2v[3, 0]
    out[...] = 1.0 / (1.0 + jnp.exp(-z3))


def _full(shape):
    return pl.BlockSpec(shape, lambda i: (0,) * len(shape))


@functools.cache
def _mlp(cs):
    return pl.pallas_call(
        _mlp_body,
        grid=(cs // BB,),
        in_specs=[
            pl.BlockSpec((BB, EMB), lambda i: (i, 0)),
            pl.BlockSpec((BB, EMB), lambda i: (i, 0)),
            _full((256, 256)), _full((128, 256)), _full((64, 128)),
            _full((64, 1)),
            _full((3, 256)), _full((3, 128)), _full((4, 64)),
        ],
        out_specs=pl.BlockSpec((BB, 1), lambda i: (i, 0)),
        out_shape=jax.ShapeDtypeStruct((cs, 1), jnp.float32),
        compiler_params=pltpu.CompilerParams(
            dimension_semantics=("arbitrary",)),
    )


def kernel(user_ids, content_ids, user_table, content_table,
           W0, b0, g0, beta0, rm0, rv0,
           W1, b1, g1, beta1, rm1, rv1,
           W2, b2, g2, beta2, rm2, rv2,
           W3, b3):
    uid = user_ids.astype(jnp.int32)
    cid = content_ids.astype(jnp.int32)
    s0 = g0 * lax.rsqrt(rv0 + EPS)
    s1 = g1 * lax.rsqrt(rv1 + EPS)
    s2 = g2 * lax.rsqrt(rv2 + EPS)
    p0 = jnp.stack([b0, s0, beta0 - rm0 * s0])
    p1 = jnp.stack([b1, s1, beta1 - rm1 * s1])
    p2 = jnp.stack([b2, s2, beta2 - rm2 * s2, jnp.broadcast_to(b3, (64,))])
    mlp_args = (W0, W1, W2, W3.reshape(64, 1), p0, p1, p2)
    scores = []
    for c in range(NSPLIT):
        ue, ce = _gather(CS, c)(uid, cid, user_table, content_table)
        scores.append(_mlp(CS)(ue, ce, *mlp_args))
    out = jnp.concatenate(scores) if NSPLIT > 1 else scores[0]
    return out.reshape(BATCH)


# speedup vs baseline: 1.1158x; 1.1042x over previous
import functools

import jax
import jax.numpy as jnp
from jax import lax
from jax.experimental import pallas as pl
from jax.experimental.pallas import tpu as pltpu
from jax.experimental.pallas import tpu_sc as plsc

BATCH = 16384
EMB = 128
EPS = 1e-5

NC = 2
NS = 16
NW = NC * NS
CH = 128

NSPLIT = 2
CS = BATCH // NSPLIT
BB = 2048


def _make_gather_body(cs, chunk):
    rpw = cs // NW
    nch = rpw // CH

    def body(uid_hbm, cid_hbm, utab_hbm, ctab_hbm, ue_hbm, ce_hbm,
             idx_v, rows_a, rows_b, sem_a, sem_b):
        wid = lax.axis_index("s") * NC + lax.axis_index("c")
        base = wid * rpw
        for tab_hbm, ids_hbm, out_hbm in ((utab_hbm, uid_hbm, ue_hbm),
                                          (ctab_hbm, cid_hbm, ce_hbm)):
            for j in range(nch):
                pltpu.sync_copy(
                    ids_hbm.at[pl.ds(chunk * cs + base + j * CH, CH)],
                    idx_v.at[j])
            pltpu.async_copy(tab_hbm.at[idx_v.at[0]], rows_a, sem_a)
            for j in range(nch):
                cur, nxt = (rows_a, rows_b) if j % 2 == 0 else (rows_b, rows_a)
                cur_s, nxt_s = (sem_a, sem_b) if j % 2 == 0 else (sem_b, sem_a)
                if j + 1 < nch:
                    pltpu.async_copy(tab_hbm.at[idx_v.at[j + 1]], nxt, nxt_s)
                pltpu.make_async_copy(tab_hbm.at[idx_v.at[j]], cur, cur_s).wait()
                pltpu.sync_copy(cur, out_hbm.at[pl.ds(base + j * CH, CH)])

    return body


@functools.cache
def _gather(cs, chunk):
    nch = cs // NW // CH
    return pl.kernel(
        _make_gather_body(cs, chunk),
        out_type=(jax.ShapeDtypeStruct((cs, EMB), jnp.float32),
                  jax.ShapeDtypeStruct((cs, EMB), jnp.float32)),
        mesh=plsc.VectorSubcoreMesh(core_axis_name="c", subcore_axis_name="s",
                                    num_cores=NC, num_subcores=NS),
        scratch_types=(pltpu.VMEM((nch, CH), jnp.int32),
                       pltpu.VMEM((CH, EMB), jnp.float32),
                       pltpu.VMEM((CH, EMB), jnp.float32),
                       pltpu.SemaphoreType.DMA,
                       pltpu.SemaphoreType.DMA),
    )


_DN_T = (((1,), (1,)), ((), ()))


def _mlp_body(ue, ce, w0, w1, w2, p0, p1, p2, out):
    bf = jnp.bfloat16
    w0v = w0[...].astype(bf)
    p0v, p1v, p2v = p0[...], p1[...], p2[...]
    z0 = (lax.dot_general(ue[...].astype(bf), w0v[:, :EMB], _DN_T,
                          preferred_element_type=jnp.float32)
          + lax.dot_general(ce[...].astype(bf), w0v[:, EMB:], _DN_T,
                            preferred_element_type=jnp.float32)
          + p0v[0:1])
    y0 = jnp.maximum(z0, 0.0) * p0v[1:2] + p0v[2:3]
    z1 = lax.dot_general(y0.astype(bf), w1[...].astype(bf), _DN_T,
                         preferred_element_type=jnp.float32) + p1v[0:1]
    y1 = jnp.maximum(z1, 0.0) * p1v[1:2] + p1v[2:3]
    z2 = lax.dot_general(y1.astype(bf), w2[...].astype(bf), _DN_T,
                         preferred_element_type=jnp.float32) + p2v[0:1]
    y2 = jnp.maximum(z2, 0.0) * p2v[1:2] + p2v[2:3]
    z3 = lax.dot_general(p2v[3:4].astype(bf), y2.astype(bf), _DN_T,
                         preferred_element_type=jnp.float32) + p2v[4, 0]
    out[...] = (1.0 / (1.0 + jnp.exp(-z3))).reshape(1, 1, -1)


def _full(shape):
    return pl.BlockSpec(shape, lambda i: (0,) * len(shape))


@functools.cache
def _mlp(cs):
    return pl.pallas_call(
        _mlp_body,
        grid=(cs // BB,),
        in_specs=[
            pl.BlockSpec((BB, EMB), lambda i: (i, 0)),
            pl.BlockSpec((BB, EMB), lambda i: (i, 0)),
            _full((256, 256)), _full((128, 256)), _full((64, 128)),
            _full((3, 256)), _full((3, 128)), _full((5, 64)),
        ],
        out_specs=pl.BlockSpec((1, 1, BB), lambda i: (i, 0, 0)),
        out_shape=jax.ShapeDtypeStruct((cs // BB, 1, BB), jnp.float32),
        compiler_params=pltpu.CompilerParams(
            dimension_semantics=("arbitrary",)),
    )


def kernel(user_ids, content_ids, user_table, content_table,
           W0, b0, g0, beta0, rm0, rv0,
           W1, b1, g1, beta1, rm1, rv1,
           W2, b2, g2, beta2, rm2, rv2,
           W3, b3):
    uid = user_ids.astype(jnp.int32)
    cid = content_ids.astype(jnp.int32)
    s0 = g0 * lax.rsqrt(rv0 + EPS)
    s1 = g1 * lax.rsqrt(rv1 + EPS)
    s2 = g2 * lax.rsqrt(rv2 + EPS)
    p0 = jnp.stack([b0, s0, beta0 - rm0 * s0])
    p1 = jnp.stack([b1, s1, beta1 - rm1 * s1])
    p2 = jnp.stack([b2, s2, beta2 - rm2 * s2, W3[0],
                    jnp.broadcast_to(b3, (64,))])
    mlp_args = (W0, W1, W2, p0, p1, p2)
    scores = []
    for c in range(NSPLIT):
        ue, ce = _gather(CS, c)(uid, cid, user_table, content_table)
        scores.append(_mlp(CS)(ue, ce, *mlp_args).reshape(CS))
    out = jnp.concatenate(scores) if NSPLIT > 1 else scores[0]
    return out


# speedup vs baseline: 1.1217x; 1.0053x over previous
import functools

import jax
import jax.numpy as jnp
from jax import lax
from jax.experimental import pallas as pl
from jax.experimental.pallas import tpu as pltpu
from jax.experimental.pallas import tpu_sc as plsc

BATCH = 16384
EMB = 128
EPS = 1e-5

NC = 2
NS = 16
NW = NC * NS
CH = 128

NSPLIT = 2
CS = BATCH // NSPLIT
BB = 4096


def _make_gather_body(cs, chunk):
    rpw = cs // NW
    nch = rpw // CH

    def body(uid_hbm, cid_hbm, utab_hbm, ctab_hbm, ue_hbm, ce_hbm,
             idx_v, rows_a, rows_b, sem_a, sem_b):
        wid = lax.axis_index("s") * NC + lax.axis_index("c")
        base = wid * rpw
        for tab_hbm, ids_hbm, out_hbm in ((utab_hbm, uid_hbm, ue_hbm),
                                          (ctab_hbm, cid_hbm, ce_hbm)):
            for j in range(nch):
                pltpu.sync_copy(
                    ids_hbm.at[pl.ds(chunk * cs + base + j * CH, CH)],
                    idx_v.at[j])
            pltpu.async_copy(tab_hbm.at[idx_v.at[0]], rows_a, sem_a)
            for j in range(nch):
                cur, nxt = (rows_a, rows_b) if j % 2 == 0 else (rows_b, rows_a)
                cur_s, nxt_s = (sem_a, sem_b) if j % 2 == 0 else (sem_b, sem_a)
                if j + 1 < nch:
                    pltpu.async_copy(tab_hbm.at[idx_v.at[j + 1]], nxt, nxt_s)
                pltpu.make_async_copy(tab_hbm.at[idx_v.at[j]], cur, cur_s).wait()
                pltpu.sync_copy(cur, out_hbm.at[pl.ds(base + j * CH, CH)])

    return body


@functools.cache
def _gather(cs, chunk):
    nch = cs // NW // CH
    return pl.kernel(
        _make_gather_body(cs, chunk),
        out_type=(jax.ShapeDtypeStruct((cs, EMB), jnp.float32),
                  jax.ShapeDtypeStruct((cs, EMB), jnp.float32)),
        mesh=plsc.VectorSubcoreMesh(core_axis_name="c", subcore_axis_name="s",
                                    num_cores=NC, num_subcores=NS),
        scratch_types=(pltpu.VMEM((nch, CH), jnp.int32),
                       pltpu.VMEM((CH, EMB), jnp.float32),
                       pltpu.VMEM((CH, EMB), jnp.float32),
                       pltpu.SemaphoreType.DMA,
                       pltpu.SemaphoreType.DMA),
    )


_DN_T = (((1,), (1,)), ((), ()))


def _mlp_body(ue, ce, w0, w1, w2, p0, p1, p2, out):
    bf = jnp.bfloat16
    w0v = w0[...].astype(bf)
    p0v, p1v, p2v = p0[...], p1[...], p2[...]
    z0 = (lax.dot_general(ue[...].astype(bf), w0v[:, :EMB], _DN_T,
                          preferred_element_type=jnp.float32)
          + lax.dot_general(ce[...].astype(bf), w0v[:, EMB:], _DN_T,
                            preferred_element_type=jnp.float32)
          + p0v[0:1])
    y0 = jnp.maximum(z0, 0.0) * p0v[1:2] + p0v[2:3]
    z1 = lax.dot_general(y0.astype(bf), w1[...].astype(bf), _DN_T,
                         preferred_element_type=jnp.float32) + p1v[0:1]
    y1 = jnp.maximum(z1, 0.0) * p1v[1:2] + p1v[2:3]
    z2 = lax.dot_general(y1.astype(bf), w2[...].astype(bf), _DN_T,
                         preferred_element_type=jnp.float32) + p2v[0:1]
    y2 = jnp.maximum(z2, 0.0) * p2v[1:2] + p2v[2:3]
    z3 = lax.dot_general(p2v[3:4].astype(bf), y2.astype(bf), _DN_T,
                         preferred_element_type=jnp.float32) + p2v[4, 0]
    out[...] = (1.0 / (1.0 + jnp.exp(-z3))).reshape(1, 1, -1)


def _full(shape):
    return pl.BlockSpec(shape, lambda i: (0,) * len(shape))


@functools.cache
def _mlp(cs):
    return pl.pallas_call(
        _mlp_body,
        grid=(cs // BB,),
        in_specs=[
            pl.BlockSpec((BB, EMB), lambda i: (i, 0)),
            pl.BlockSpec((BB, EMB), lambda i: (i, 0)),
            _full((256, 256)), _full((128, 256)), _full((64, 128)),
            _full((3, 256)), _full((3, 128)), _full((5, 64)),
        ],
        out_specs=pl.BlockSpec((1, 1, BB), lambda i: (i, 0, 0)),
        out_shape=jax.ShapeDtypeStruct((cs // BB, 1, BB), jnp.float32),
        compiler_params=pltpu.CompilerParams(
            dimension_semantics=("arbitrary",)),
    )


def kernel(user_ids, content_ids, user_table, content_table,
           W0, b0, g0, beta0, rm0, rv0,
           W1, b1, g1, beta1, rm1, rv1,
           W2, b2, g2, beta2, rm2, rv2,
           W3, b3):
    uid = user_ids.astype(jnp.int32)
    cid = content_ids.astype(jnp.int32)
    s0 = g0 * lax.rsqrt(rv0 + EPS)
    s1 = g1 * lax.rsqrt(rv1 + EPS)
    s2 = g2 * lax.rsqrt(rv2 + EPS)
    p0 = jnp.stack([b0, s0, beta0 - rm0 * s0])
    p1 = jnp.stack([b1, s1, beta1 - rm1 * s1])
    p2 = jnp.stack([b2, s2, beta2 - rm2 * s2, W3[0],
                    jnp.broadcast_to(b3, (64,))])
    mlp_args = (W0, W1, W2, p0, p1, p2)
    scores = []
    for c in range(NSPLIT):
        ue, ce = _gather(CS, c)(uid, cid, user_table, content_table)
        scores.append(_mlp(CS)(ue, ce, *mlp_args).reshape(CS))
    out = jnp.concatenate(scores) if NSPLIT > 1 else scores[0]
    return out
